# baseline (speedup 1.0000x reference)
import jax
import jax.numpy as jnp
from jax.experimental import pallas as pl
from jax.experimental.pallas import tpu as pltpu

_NUM_EMB = 128
_BB = 2


def _vq_block(x_ref, t_ref, q_ref, loss_ref):
    tab = t_ref[...]
    e2 = jnp.sum(tab * tab, axis=1, keepdims=True)
    tabm2 = -2.0 * tab
    part = jnp.zeros((), jnp.float32)
    iota_k = jax.lax.broadcasted_iota(jnp.int32, (_NUM_EMB, x_ref.shape[2]), 0)
    krow = jax.lax.broadcasted_iota(jnp.int32, (1, _NUM_EMB), 1)
    w = jax.lax.bitcast_convert_type(
        jax.lax.shift_left(191 - krow, 23), jnp.float32).astype(jnp.bfloat16)
    for i in range(_BB):
        xb = x_ref[i]
        m2s = jax.lax.dot_general(tabm2, xb, (((1,), (0,)), ((), ())),
                                  preferred_element_type=jnp.float32)
        sqx = jnp.sum(xb * xb, axis=0, keepdims=True)
        d = (sqx + m2s) + e2
        mind = jnp.min(d, axis=0, keepdims=True)
        eqf = (d == mind).astype(jnp.bfloat16)
        m = jax.lax.dot_general(w, eqf, (((1,), (0,)), ((), ())),
                                preferred_element_type=jnp.float32)
        first_k = 191 - jax.lax.shift_right_logical(
            jax.lax.bitcast_convert_type(m, jnp.int32), 23)
        onehot = (iota_k == first_k).astype(jnp.float32)
        q = jax.lax.dot_general(tab, onehot, (((0,), (0,)), ((), ())),
                                preferred_element_type=jnp.float32)
        q_ref[i] = q
        part += jnp.sum(mind)
    loss_ref[...] = part.reshape(1, 1, 1)


def kernel(x, table):
    B, C, H, W = x.shape
    xv = x.reshape(B, C, H * W)
    nblk = B // _BB
    q, loss_parts = pl.pallas_call(
        _vq_block,
        grid=(nblk,),
        in_specs=[
            pl.BlockSpec((_BB, C, H * W), lambda b: (b, 0, 0)),
            pl.BlockSpec((_NUM_EMB, C), lambda b: (0, 0)),
        ],
        out_specs=[
            pl.BlockSpec((_BB, C, H * W), lambda b: (b, 0, 0)),
            pl.BlockSpec((1, 1, 1), lambda b: (b, 0, 0)),
        ],
        out_shape=[
            jax.ShapeDtypeStruct((B, C, H * W), jnp.float32),
            jax.ShapeDtypeStruct((nblk, 1, 1), jnp.float32),
        ],
        compiler_params=pltpu.CompilerParams(
            dimension_semantics=("parallel",)),
    )(xv, table)
    loss = jnp.sum(loss_parts) * (1.25 / (B * C * H * W))
    return q.reshape(B, C, H, W), loss


# speedup vs baseline: 1.0323x; 1.0323x over previous
import jax
import jax.numpy as jnp
from jax.experimental import pallas as pl
from jax.experimental.pallas import tpu as pltpu

_NUM_EMB = 128
_BB = 4


def _vq_block(x_ref, t_ref, q_ref, loss_ref):
    tab = t_ref[...]
    e2 = jnp.sum(tab * tab, axis=1, keepdims=True)
    tabm2 = -2.0 * tab
    part = jnp.zeros((), jnp.float32)
    iota_k = jax.lax.broadcasted_iota(jnp.int32, (_NUM_EMB, x_ref.shape[2]), 0)
    krow = jax.lax.broadcasted_iota(jnp.int32, (1, _NUM_EMB), 1)
    w = jax.lax.bitcast_convert_type(
        jax.lax.shift_left(191 - krow, 23), jnp.float32).astype(jnp.bfloat16)
    for i in range(_BB):
        xb = x_ref[i]
        m2s = jax.lax.dot_general(tabm2, xb, (((1,), (0,)), ((), ())),
                                  preferred_element_type=jnp.float32)
        sqx = jnp.sum(xb * xb, axis=0, keepdims=True)
        d = (sqx + m2s) + e2
        mind = jnp.min(d, axis=0, keepdims=True)
        eqf = (d == mind).astype(jnp.bfloat16)
        m = jax.lax.dot_general(w, eqf, (((1,), (0,)), ((), ())),
                                preferred_element_type=jnp.float32)
        first_k = 191 - jax.lax.shift_right_logical(
            jax.lax.bitcast_convert_type(m, jnp.int32), 23)
        onehot = (iota_k == first_k).astype(jnp.float32)
        q = jax.lax.dot_general(tab, onehot, (((0,), (0,)), ((), ())),
                                preferred_element_type=jnp.float32)
        q_ref[i] = q
        part += jnp.sum(mind)
    loss_ref[...] = part.reshape(1, 1, 1)


def kernel(x, table):
    B, C, H, W = x.shape
    xv = x.reshape(B, C, H * W)
    nblk = B // _BB
    q, loss_parts = pl.pallas_call(
        _vq_block,
        grid=(nblk,),
        in_specs=[
            pl.BlockSpec((_BB, C, H * W), lambda b: (b, 0, 0)),
            pl.BlockSpec((_NUM_EMB, C), lambda b: (0, 0)),
        ],
        out_specs=[
            pl.BlockSpec((_BB, C, H * W), lambda b: (b, 0, 0)),
            pl.BlockSpec((1, 1, 1), lambda b: (b, 0, 0)),
        ],
        out_shape=[
            jax.ShapeDtypeStruct((B, C, H * W), jnp.float32),
            jax.ShapeDtypeStruct((nblk, 1, 1), jnp.float32),
        ],
        compiler_params=pltpu.CompilerParams(
            dimension_semantics=("parallel",)),
    )(xv, table)
    loss = jnp.sum(loss_parts) * (1.25 / (B * C * H * W))
    return q.reshape(B, C, H, W), loss
